# baseline (speedup 1.0000x reference)
import functools

import jax
import jax.numpy as jnp
from jax import lax
from jax.experimental import pallas as pl
from jax.experimental.pallas import tpu as pltpu
from jax.experimental.pallas import tpu_sc as plsc

_N = 10000
_D = 128
_H = 8
_CH = 16
_E = 320000

_NC, _NS, _L = 2, 16, 16
_NW = _NC * _NS
_NP = 10240
_RPT = _NP // _NS
_K = 64
_NCHUNK = 162
_EPAD = _NW * _NCHUNK * _K
_EALLOC = _EPAD + _K
_NB = 3


def _sc_edge_build():
    mesh = plsc.VectorSubcoreMesh(
        core_axis_name="c", subcore_axis_name="s",
        num_cores=_NC, num_subcores=_NS)

    @functools.partial(
        pl.kernel,
        out_type=(jax.ShapeDtypeStruct((_NC, _NP, _D), jnp.float32),
                  jax.ShapeDtypeStruct((_NC, _NP, _L), jnp.float32)),
        mesh=mesh,
        compiler_params=pltpu.CompilerParams(use_tc_tiling_on_sc=False),
        scratch_types=[
            pltpu.VMEM((_NB, _K), jnp.int32),
            pltpu.VMEM((_NB, _K), jnp.int32),
            pltpu.VMEM((_NB, _K, _L), jnp.float32),
            pltpu.VMEM((_NB, _K, _L), jnp.float32),
            pltpu.VMEM((_NB, _K, _L), jnp.float32),
            pltpu.VMEM((_NB, _K, _D), jnp.float32),
            pltpu.VMEM_SHARED((_NP, _D), jnp.float32),
            pltpu.VMEM_SHARED((_NP, _L), jnp.float32),
            pltpu.SemaphoreType.DMA((_NB,)),
            pltpu.SemaphoreType.DMA((_NB,)),
        ],
    )
    def sc_edge(src_hbm, dst_hbm, zd_hbm, zl_hbm, as_hbm, ad_hbm, h_hbm,
                acc_out, den_out,
                srcv, dstv, asv, adv, ev, hv, acc_sh, den_sh,
                gsem, ssem):
        cid = lax.axis_index("c")
        sid = lax.axis_index("s")
        wid = cid * _NS + sid
        row0 = sid * _RPT
        tbase = wid * (_NCHUNK * _K)

        pltpu.sync_copy(zd_hbm, hv.at[0])
        pltpu.sync_copy(zl_hbm, ev.at[0])
        for t in range(_RPT // _K):
            pltpu.sync_copy(hv.at[0], acc_sh.at[pl.ds(row0 + t * _K, _K)])
            pltpu.sync_copy(ev.at[0], den_sh.at[pl.ds(row0 + t * _K, _K)])
        plsc.subcore_barrier()

        def issue_gathers(g, b):
            base = tbase + g * _K
            pltpu.sync_copy(src_hbm.at[pl.ds(base, _K)], srcv.at[b])
            pltpu.sync_copy(dst_hbm.at[pl.ds(base, _K)], dstv.at[b])
            pltpu.async_copy(as_hbm.at[srcv.at[b]], asv.at[b], gsem.at[b])
            pltpu.async_copy(ad_hbm.at[dstv.at[b]], adv.at[b], gsem.at[b])
            pltpu.async_copy(h_hbm.at[srcv.at[b]], hv.at[b], gsem.at[b])

        def wait_gathers(b):
            pltpu.make_async_copy(
                as_hbm.at[srcv.at[b]], asv.at[b], gsem.at[b]).wait()
            pltpu.make_async_copy(
                ad_hbm.at[dstv.at[b]], adv.at[b], gsem.at[b]).wait()
            pltpu.make_async_copy(
                h_hbm.at[srcv.at[b]], hv.at[b], gsem.at[b]).wait()

        def wait_scatters(b):
            pltpu.make_async_copy(
                ev.at[b], den_sh.at[dstv.at[b]], ssem.at[b]).wait()
            pltpu.make_async_copy(
                hv.at[b], acc_sh.at[dstv.at[b]], ssem.at[b]).wait()

        issue_gathers(0, 0)

        def superstep(p, carry):
            for b in range(_NB):
                g = _NB * p + b
                nxt = (b + 1) % _NB
                @pl.when(g >= _NB - 1)
                def _():
                    wait_scatters(nxt)
                issue_gathers(g + 1, nxt)
                wait_gathers(b)

                def edge_body(i, c):
                    a = asv[b, i] + adv[b, i]
                    a = jnp.where(a > 0, a, 0.2 * a)
                    erow = jnp.exp(a)
                    ev[b, i] = erow
                    for hh in range(_H):
                        sv = jnp.full((_L,), erow[hh], jnp.float32)
                        hv[b, i, pl.ds(hh * _L, _L)] = (
                            hv[b, i, pl.ds(hh * _L, _L)] * sv)
                    return c
                lax.fori_loop(0, _K, edge_body, 0, unroll=2)

                pltpu.async_copy(
                    ev.at[b], den_sh.at[dstv.at[b]], ssem.at[b], add=True)
                pltpu.async_copy(
                    hv.at[b], acc_sh.at[dstv.at[b]], ssem.at[b], add=True)
            return carry
        lax.fori_loop(0, _NCHUNK // _NB, superstep, 0)

        wait_gathers(0)
        wait_scatters(1)
        wait_scatters(2)
        plsc.subcore_barrier()

        for t in range(_RPT // _K):
            pltpu.sync_copy(acc_sh.at[pl.ds(row0 + t * _K, _K)], hv.at[0])
            pltpu.sync_copy(hv.at[0],
                            acc_out.at[cid, pl.ds(row0 + t * _K, _K)])
            pltpu.sync_copy(den_sh.at[pl.ds(row0 + t * _K, _K)], ev.at[0])
            pltpu.sync_copy(ev.at[0],
                            den_out.at[cid, pl.ds(row0 + t * _K, _K)])

    return sc_edge


_SC_EDGE = _sc_edge_build()


def _tc_dense(xin, W, As, Ad):
    def body(x_ref, w_ref, as_ref, ad_ref, h_ref, s_ref, d_ref):
        h = jnp.dot(x_ref[...], w_ref[...],
                    preferred_element_type=jnp.float32)
        h_ref[...] = h
        s_ref[...] = jnp.dot(h, as_ref[...],
                             preferred_element_type=jnp.float32)
        d_ref[...] = jnp.dot(h, ad_ref[...],
                             preferred_element_type=jnp.float32)
    return pl.pallas_call(
        body,
        out_shape=(jax.ShapeDtypeStruct((_NP, _D), jnp.float32),
                   jax.ShapeDtypeStruct((_NP, _L), jnp.float32),
                   jax.ShapeDtypeStruct((_NP, _L), jnp.float32)),
    )(xin, W, As, Ad)


def _tc_combine_dense(acc, den, R, bias, W, As, Ad):
    def body(a_ref, d_ref, r_ref, b_ref, w_ref, as_ref, ad_ref,
             h_ref, s_ref, dt_ref):
        s = a_ref[0] + a_ref[1]
        dn = d_ref[0] + d_ref[1]
        dd = jnp.dot(dn, r_ref[...], preferred_element_type=jnp.float32)
        o = s / (dd + 1e-16) + b_ref[...]
        o = jnp.where(o > 0, o, jnp.exp(o) - 1.0)
        h = jnp.dot(o, w_ref[...], preferred_element_type=jnp.float32)
        h_ref[...] = h
        s_ref[...] = jnp.dot(h, as_ref[...],
                             preferred_element_type=jnp.float32)
        dt_ref[...] = jnp.dot(h, ad_ref[...],
                              preferred_element_type=jnp.float32)
    return pl.pallas_call(
        body,
        out_shape=(jax.ShapeDtypeStruct((_NP, _D), jnp.float32),
                   jax.ShapeDtypeStruct((_NP, _L), jnp.float32),
                   jax.ShapeDtypeStruct((_NP, _L), jnp.float32)),
    )(acc, den, R, bias, W, As, Ad)


def _tc_combine_final(acc, den, R, bias):
    def body(a_ref, d_ref, r_ref, b_ref, o_ref):
        s = a_ref[0, :_N] + a_ref[1, :_N]
        dn = d_ref[0, :_N] + d_ref[1, :_N]
        dd = jnp.dot(dn, r_ref[...], preferred_element_type=jnp.float32)
        o_ref[...] = s / (dd + 1e-16) + b_ref[...]
    return pl.pallas_call(
        body,
        out_shape=jax.ShapeDtypeStruct((_N, _D), jnp.float32),
    )(acc, den, R, bias)


def kernel(x, edge_index, W1, att_src1, att_dst1, b1,
           W2, att_src2, att_dst2, b2):
    f32 = jnp.float32
    loop = jnp.arange(_N, dtype=edge_index.dtype)
    npad = _EALLOC - (_E + _N)
    padv = jnp.full((npad,), _N, edge_index.dtype)
    src = jnp.concatenate([edge_index[0], loop, padv]).astype(jnp.int32)
    dst = jnp.concatenate([edge_index[1], loop, padv]).astype(jnp.int32)

    x_pad = jnp.zeros((_NP, _D), f32).at[:_N].set(x)

    eye8 = jnp.eye(_H, dtype=f32)
    def blockdiag(att):
        m = (att[0][:, :, None] * eye8[:, None, :]).reshape(_D, _H)
        return jnp.concatenate([m, jnp.zeros((_D, _H), f32)], axis=1)
    As1 = blockdiag(att_src1)
    Ad1 = blockdiag(att_dst1)
    As2 = jnp.broadcast_to(att_src2[0, 0][:, None], (_D, _L)).astype(f32)
    Ad2 = jnp.broadcast_to(att_dst2[0, 0][:, None], (_D, _L)).astype(f32)

    R = jnp.concatenate(
        [jnp.kron(eye8, jnp.ones((1, _CH), f32)),
         jnp.zeros((_H, _D), f32)], axis=0)

    zd = jnp.zeros((_K, _D), f32)
    zl = jnp.zeros((_K, _L), f32)
    b1r = b1.reshape(1, _D)
    b2r = b2.reshape(1, _D)

    h1, s1, d1 = _tc_dense(x_pad, W1, As1, Ad1)
    acc1, den1 = _SC_EDGE(src, dst, zd, zl, s1, d1, h1)
    h2, s2, d2 = _tc_combine_dense(acc1, den1, R, b1r, W2, As2, Ad2)
    acc2, den2 = _SC_EDGE(src, dst, zd, zl, s2, d2, h2)
    return _tc_combine_final(acc2, den2, R, b2r)


# speedup vs baseline: 1.0037x; 1.0037x over previous
import functools

import jax
import jax.numpy as jnp
from jax import lax
from jax.experimental import pallas as pl
from jax.experimental.pallas import tpu as pltpu
from jax.experimental.pallas import tpu_sc as plsc

_N = 10000
_D = 128
_H = 8
_CH = 16
_E = 320000

_NC, _NS, _L = 2, 16, 16
_NW = _NC * _NS
_NP = 10240
_RPT = _NP // _NS
_K = 64
_NCHUNK = 162
_EPAD = _NW * _NCHUNK * _K
_EALLOC = _EPAD + _K
_NB = 3


def _sc_edge_build():
    mesh = plsc.VectorSubcoreMesh(
        core_axis_name="c", subcore_axis_name="s",
        num_cores=_NC, num_subcores=_NS)

    @functools.partial(
        pl.kernel,
        out_type=(jax.ShapeDtypeStruct((_NC, _NP, _D), jnp.float32),
                  jax.ShapeDtypeStruct((_NC, _NP, _L), jnp.float32)),
        mesh=mesh,
        compiler_params=pltpu.CompilerParams(use_tc_tiling_on_sc=False),
        scratch_types=[
            pltpu.VMEM((_NB, _K), jnp.int32),
            pltpu.VMEM((_NB, _K), jnp.int32),
            pltpu.VMEM((_NB, _K, _L), jnp.float32),
            pltpu.VMEM((_NB, _K, _L), jnp.float32),
            pltpu.VMEM((_NB, _K, _L), jnp.float32),
            pltpu.VMEM((_NB, _K, _D), jnp.float32),
            pltpu.VMEM_SHARED((_NP, _D), jnp.float32),
            pltpu.VMEM_SHARED((_NP, _L), jnp.float32),
            pltpu.SemaphoreType.DMA((_NB,)),
            pltpu.SemaphoreType.DMA((_NB,)),
        ],
    )
    def sc_edge(src_hbm, dst_hbm, zd_hbm, zl_hbm, as_hbm, ad_hbm, h_hbm,
                acc_out, den_out,
                srcv, dstv, asv, adv, ev, hv, acc_sh, den_sh,
                gsem, ssem):
        cid = lax.axis_index("c")
        sid = lax.axis_index("s")
        wid = cid * _NS + sid
        row0 = sid * _RPT
        tbase = wid * (_NCHUNK * _K)

        pltpu.sync_copy(zd_hbm, hv.at[0])
        pltpu.sync_copy(zl_hbm, ev.at[0])
        for t in range(_RPT // _K):
            pltpu.sync_copy(hv.at[0], acc_sh.at[pl.ds(row0 + t * _K, _K)])
            pltpu.sync_copy(ev.at[0], den_sh.at[pl.ds(row0 + t * _K, _K)])
        plsc.subcore_barrier()

        def issue_gathers(g, b):
            base = tbase + g * _K
            pltpu.sync_copy(src_hbm.at[pl.ds(base, _K)], srcv.at[b])
            pltpu.sync_copy(dst_hbm.at[pl.ds(base, _K)], dstv.at[b])
            pltpu.async_copy(as_hbm.at[srcv.at[b]], asv.at[b], gsem.at[b])
            pltpu.async_copy(ad_hbm.at[dstv.at[b]], adv.at[b], gsem.at[b])
            pltpu.async_copy(h_hbm.at[srcv.at[b]], hv.at[b], gsem.at[b])

        def wait_gathers(b):
            pltpu.make_async_copy(
                as_hbm.at[srcv.at[b]], asv.at[b], gsem.at[b]).wait()
            pltpu.make_async_copy(
                ad_hbm.at[dstv.at[b]], adv.at[b], gsem.at[b]).wait()
            pltpu.make_async_copy(
                h_hbm.at[srcv.at[b]], hv.at[b], gsem.at[b]).wait()

        def wait_scatters(b):
            pltpu.make_async_copy(
                ev.at[b], den_sh.at[dstv.at[b]], ssem.at[b]).wait()
            pltpu.make_async_copy(
                hv.at[b], acc_sh.at[dstv.at[b]], ssem.at[b]).wait()

        issue_gathers(0, 0)

        def superstep(p, carry):
            for b in range(_NB):
                g = _NB * p + b
                nxt = (b + 1) % _NB
                @pl.when(g >= _NB - 1)
                def _():
                    wait_scatters(nxt)
                issue_gathers(g + 1, nxt)
                wait_gathers(b)

                def edge_body(i, c):
                    a = asv[b, i] + adv[b, i]
                    a = jnp.where(a > 0, a, 0.2 * a)
                    erow = jnp.exp(a)
                    ev[b, i] = erow
                    for hh in range(_H):
                        sv = jnp.full((_L,), erow[hh], jnp.float32)
                        hv[b, i, pl.ds(hh * _L, _L)] = (
                            hv[b, i, pl.ds(hh * _L, _L)] * sv)
                    return c
                lax.fori_loop(0, _K, edge_body, 0)

                pltpu.async_copy(
                    ev.at[b], den_sh.at[dstv.at[b]], ssem.at[b], add=True)
                pltpu.async_copy(
                    hv.at[b], acc_sh.at[dstv.at[b]], ssem.at[b], add=True)
            return carry
        lax.fori_loop(0, _NCHUNK // _NB, superstep, 0)

        wait_gathers(0)
        wait_scatters(1)
        wait_scatters(2)
        plsc.subcore_barrier()

        for t in range(_RPT // _K):
            pltpu.sync_copy(acc_sh.at[pl.ds(row0 + t * _K, _K)], hv.at[0])
            pltpu.sync_copy(hv.at[0],
                            acc_out.at[cid, pl.ds(row0 + t * _K, _K)])
            pltpu.sync_copy(den_sh.at[pl.ds(row0 + t * _K, _K)], ev.at[0])
            pltpu.sync_copy(ev.at[0],
                            den_out.at[cid, pl.ds(row0 + t * _K, _K)])

    return sc_edge


_SC_EDGE = _sc_edge_build()


def _tc_dense(xin, W, As, Ad):
    def body(x_ref, w_ref, as_ref, ad_ref, h_ref, s_ref, d_ref):
        h = jnp.dot(x_ref[...], w_ref[...],
                    preferred_element_type=jnp.float32)
        h_ref[...] = h
        s_ref[...] = jnp.dot(h, as_ref[...],
                             preferred_element_type=jnp.float32)
        d_ref[...] = jnp.dot(h, ad_ref[...],
                             preferred_element_type=jnp.float32)
    return pl.pallas_call(
        body,
        out_shape=(jax.ShapeDtypeStruct((_NP, _D), jnp.float32),
                   jax.ShapeDtypeStruct((_NP, _L), jnp.float32),
                   jax.ShapeDtypeStruct((_NP, _L), jnp.float32)),
    )(xin, W, As, Ad)


def _tc_combine_dense(acc, den, R, bias, W, As, Ad):
    def body(a_ref, d_ref, r_ref, b_ref, w_ref, as_ref, ad_ref,
             h_ref, s_ref, dt_ref):
        s = a_ref[0] + a_ref[1]
        dn = d_ref[0] + d_ref[1]
        dd = jnp.dot(dn, r_ref[...], preferred_element_type=jnp.float32)
        o = s / (dd + 1e-16) + b_ref[...]
        o = jnp.where(o > 0, o, jnp.exp(o) - 1.0)
        h = jnp.dot(o, w_ref[...], preferred_element_type=jnp.float32)
        h_ref[...] = h
        s_ref[...] = jnp.dot(h, as_ref[...],
                             preferred_element_type=jnp.float32)
        dt_ref[...] = jnp.dot(h, ad_ref[...],
                              preferred_element_type=jnp.float32)
    return pl.pallas_call(
        body,
        out_shape=(jax.ShapeDtypeStruct((_NP, _D), jnp.float32),
                   jax.ShapeDtypeStruct((_NP, _L), jnp.float32),
                   jax.ShapeDtypeStruct((_NP, _L), jnp.float32)),
    )(acc, den, R, bias, W, As, Ad)


def _tc_combine_final(acc, den, R, bias):
    def body(a_ref, d_ref, r_ref, b_ref, o_ref):
        s = a_ref[0, :_N] + a_ref[1, :_N]
        dn = d_ref[0, :_N] + d_ref[1, :_N]
        dd = jnp.dot(dn, r_ref[...], preferred_element_type=jnp.float32)
        o_ref[...] = s / (dd + 1e-16) + b_ref[...]
    return pl.pallas_call(
        body,
        out_shape=jax.ShapeDtypeStruct((_N, _D), jnp.float32),
    )(acc, den, R, bias)


def kernel(x, edge_index, W1, att_src1, att_dst1, b1,
           W2, att_src2, att_dst2, b2):
    f32 = jnp.float32
    loop = jnp.arange(_N, dtype=edge_index.dtype)
    npad = _EALLOC - (_E + _N)
    padv = jnp.full((npad,), _N, edge_index.dtype)
    src = jnp.concatenate([edge_index[0], loop, padv]).astype(jnp.int32)
    dst = jnp.concatenate([edge_index[1], loop, padv]).astype(jnp.int32)

    x_pad = jnp.zeros((_NP, _D), f32).at[:_N].set(x)

    eye8 = jnp.eye(_H, dtype=f32)
    def blockdiag(att):
        m = (att[0][:, :, None] * eye8[:, None, :]).reshape(_D, _H)
        return jnp.concatenate([m, jnp.zeros((_D, _H), f32)], axis=1)
    As1 = blockdiag(att_src1)
    Ad1 = blockdiag(att_dst1)
    As2 = jnp.broadcast_to(att_src2[0, 0][:, None], (_D, _L)).astype(f32)
    Ad2 = jnp.broadcast_to(att_dst2[0, 0][:, None], (_D, _L)).astype(f32)

    R = jnp.concatenate(
        [jnp.kron(eye8, jnp.ones((1, _CH), f32)),
         jnp.zeros((_H, _D), f32)], axis=0)

    zd = jnp.zeros((_K, _D), f32)
    zl = jnp.zeros((_K, _L), f32)
    b1r = b1.reshape(1, _D)
    b2r = b2.reshape(1, _D)

    h1, s1, d1 = _tc_dense(x_pad, W1, As1, Ad1)
    acc1, den1 = _SC_EDGE(src, dst, zd, zl, s1, d1, h1)
    h2, s2, d2 = _tc_combine_dense(acc1, den1, R, b1r, W2, As2, Ad2)
    acc2, den2 = _SC_EDGE(src, dst, zd, zl, s2, d2, h2)
    return _tc_combine_final(acc2, den2, R, b2r)


# speedup vs baseline: 1.1881x; 1.1838x over previous
import functools

import jax
import jax.numpy as jnp
from jax import lax
from jax.experimental import pallas as pl
from jax.experimental.pallas import tpu as pltpu
from jax.experimental.pallas import tpu_sc as plsc

_N = 10000
_D = 128
_H = 8
_CH = 16
_E = 320000

_NC, _NS, _L = 2, 16, 16
_NW = _NC * _NS
_NP = 10240
_RPT = _NP // _NS
_K = 64
_NCHUNK = 162
_EPAD = _NW * _NCHUNK * _K
_EALLOC = _EPAD + _K
_NB = 3


def _sc_edge_build():
    mesh = plsc.VectorSubcoreMesh(
        core_axis_name="c", subcore_axis_name="s",
        num_cores=_NC, num_subcores=_NS)

    @functools.partial(
        pl.kernel,
        out_type=(jax.ShapeDtypeStruct((_NC, _NP, _D), jnp.float32),
                  jax.ShapeDtypeStruct((_NC, _NP, _L), jnp.float32)),
        mesh=mesh,
        compiler_params=pltpu.CompilerParams(use_tc_tiling_on_sc=False),
        scratch_types=[
            pltpu.VMEM((_NB, _K), jnp.int32),
            pltpu.VMEM((_NB, _K), jnp.int32),
            pltpu.VMEM((_NB, _K, _L), jnp.float32),
            pltpu.VMEM((_NB, _K, _L), jnp.float32),
            pltpu.VMEM((_NB, _K, _L), jnp.float32),
            pltpu.VMEM((_NB, _K, _D), jnp.float32),
            pltpu.VMEM_SHARED((_NP, _D), jnp.float32),
            pltpu.VMEM_SHARED((_NP, _L), jnp.float32),
            pltpu.SemaphoreType.DMA((_NB,)),
            pltpu.SemaphoreType.DMA((_NB,)),
        ],
    )
    def sc_edge(src_hbm, dst_hbm, zd_hbm, zl_hbm, as_hbm, ad_hbm, h_hbm,
                acc_out, den_out,
                srcv, dstv, asv, adv, ev, hv, acc_sh, den_sh,
                gsem, ssem):
        cid = lax.axis_index("c")
        sid = lax.axis_index("s")
        wid = cid * _NS + sid
        row0 = sid * _RPT
        tbase = wid * (_NCHUNK * _K)

        pltpu.sync_copy(zd_hbm, hv.at[0])
        pltpu.sync_copy(zl_hbm, ev.at[0])
        for t in range(_RPT // _K):
            pltpu.sync_copy(hv.at[0], acc_sh.at[pl.ds(row0 + t * _K, _K)])
            pltpu.sync_copy(ev.at[0], den_sh.at[pl.ds(row0 + t * _K, _K)])
        plsc.subcore_barrier()

        def issue_gathers(g, b):
            base = tbase + g * _K
            pltpu.sync_copy(src_hbm.at[pl.ds(base, _K)], srcv.at[b])
            pltpu.sync_copy(dst_hbm.at[pl.ds(base, _K)], dstv.at[b])
            pltpu.async_copy(as_hbm.at[srcv.at[b]], asv.at[b], gsem.at[b])
            pltpu.async_copy(ad_hbm.at[dstv.at[b]], adv.at[b], gsem.at[b])
            pltpu.async_copy(h_hbm.at[srcv.at[b]], hv.at[b], gsem.at[b])

        def wait_gathers(b):
            pltpu.make_async_copy(
                as_hbm.at[srcv.at[b]], asv.at[b], gsem.at[b]).wait()
            pltpu.make_async_copy(
                ad_hbm.at[dstv.at[b]], adv.at[b], gsem.at[b]).wait()
            pltpu.make_async_copy(
                h_hbm.at[srcv.at[b]], hv.at[b], gsem.at[b]).wait()

        def wait_scatters(b):
            pltpu.make_async_copy(
                ev.at[b], den_sh.at[dstv.at[b]], ssem.at[b]).wait()
            pltpu.make_async_copy(
                hv.at[b], acc_sh.at[dstv.at[b]], ssem.at[b]).wait()

        issue_gathers(0, 0)

        def superstep(p, carry):
            for b in range(_NB):
                g = _NB * p + b
                nxt = (b + 1) % _NB
                @pl.when(g >= _NB - 1)
                def _():
                    wait_scatters(nxt)
                issue_gathers(g + 1, nxt)
                wait_gathers(b)

                def edge_e(i, c):
                    a = asv[b, i] + adv[b, i]
                    a = jnp.where(a > 0, a, 0.2 * a)
                    ev[b, i] = jnp.exp(a)
                    return c
                lax.fori_loop(0, _K, edge_e, 0)

                def edge_scale(i, c):
                    erow = ev[b, i]
                    for hh in range(_H):
                        sv = jnp.full((_L,), erow[hh], jnp.float32)
                        hv[b, i, pl.ds(hh * _L, _L)] = (
                            hv[b, i, pl.ds(hh * _L, _L)] * sv)
                    return c
                lax.fori_loop(0, _K, edge_scale, 0)

                pltpu.async_copy(
                    ev.at[b], den_sh.at[dstv.at[b]], ssem.at[b], add=True)
                pltpu.async_copy(
                    hv.at[b], acc_sh.at[dstv.at[b]], ssem.at[b], add=True)
            return carry
        lax.fori_loop(0, _NCHUNK // _NB, superstep, 0)

        wait_gathers(0)
        wait_scatters(1)
        wait_scatters(2)
        plsc.subcore_barrier()

        for t in range(_RPT // _K):
            pltpu.sync_copy(acc_sh.at[pl.ds(row0 + t * _K, _K)], hv.at[0])
            pltpu.sync_copy(hv.at[0],
                            acc_out.at[cid, pl.ds(row0 + t * _K, _K)])
            pltpu.sync_copy(den_sh.at[pl.ds(row0 + t * _K, _K)], ev.at[0])
            pltpu.sync_copy(ev.at[0],
                            den_out.at[cid, pl.ds(row0 + t * _K, _K)])

    return sc_edge


_SC_EDGE = _sc_edge_build()


def _tc_dense(xin, W, As, Ad):
    def body(x_ref, w_ref, as_ref, ad_ref, h_ref, s_ref, d_ref):
        h = jnp.dot(x_ref[...], w_ref[...],
                    preferred_element_type=jnp.float32)
        h_ref[...] = h
        s_ref[...] = jnp.dot(h, as_ref[...],
                             preferred_element_type=jnp.float32)
        d_ref[...] = jnp.dot(h, ad_ref[...],
                             preferred_element_type=jnp.float32)
    return pl.pallas_call(
        body,
        out_shape=(jax.ShapeDtypeStruct((_NP, _D), jnp.float32),
                   jax.ShapeDtypeStruct((_NP, _L), jnp.float32),
                   jax.ShapeDtypeStruct((_NP, _L), jnp.float32)),
    )(xin, W, As, Ad)


def _tc_combine_dense(acc, den, R, bias, W, As, Ad):
    def body(a_ref, d_ref, r_ref, b_ref, w_ref, as_ref, ad_ref,
             h_ref, s_ref, dt_ref):
        s = a_ref[0] + a_ref[1]
        dn = d_ref[0] + d_ref[1]
        dd = jnp.dot(dn, r_ref[...], preferred_element_type=jnp.float32)
        o = s / (dd + 1e-16) + b_ref[...]
        o = jnp.where(o > 0, o, jnp.exp(o) - 1.0)
        h = jnp.dot(o, w_ref[...], preferred_element_type=jnp.float32)
        h_ref[...] = h
        s_ref[...] = jnp.dot(h, as_ref[...],
                             preferred_element_type=jnp.float32)
        dt_ref[...] = jnp.dot(h, ad_ref[...],
                              preferred_element_type=jnp.float32)
    return pl.pallas_call(
        body,
        out_shape=(jax.ShapeDtypeStruct((_NP, _D), jnp.float32),
                   jax.ShapeDtypeStruct((_NP, _L), jnp.float32),
                   jax.ShapeDtypeStruct((_NP, _L), jnp.float32)),
    )(acc, den, R, bias, W, As, Ad)


def _tc_combine_final(acc, den, R, bias):
    def body(a_ref, d_ref, r_ref, b_ref, o_ref):
        s = a_ref[0, :_N] + a_ref[1, :_N]
        dn = d_ref[0, :_N] + d_ref[1, :_N]
        dd = jnp.dot(dn, r_ref[...], preferred_element_type=jnp.float32)
        o_ref[...] = s / (dd + 1e-16) + b_ref[...]
    return pl.pallas_call(
        body,
        out_shape=jax.ShapeDtypeStruct((_N, _D), jnp.float32),
    )(acc, den, R, bias)


def kernel(x, edge_index, W1, att_src1, att_dst1, b1,
           W2, att_src2, att_dst2, b2):
    f32 = jnp.float32
    loop = jnp.arange(_N, dtype=edge_index.dtype)
    npad = _EALLOC - (_E + _N)
    padv = jnp.full((npad,), _N, edge_index.dtype)
    src = jnp.concatenate([edge_index[0], loop, padv]).astype(jnp.int32)
    dst = jnp.concatenate([edge_index[1], loop, padv]).astype(jnp.int32)

    x_pad = jnp.zeros((_NP, _D), f32).at[:_N].set(x)

    eye8 = jnp.eye(_H, dtype=f32)
    def blockdiag(att):
        m = (att[0][:, :, None] * eye8[:, None, :]).reshape(_D, _H)
        return jnp.concatenate([m, jnp.zeros((_D, _H), f32)], axis=1)
    As1 = blockdiag(att_src1)
    Ad1 = blockdiag(att_dst1)
    As2 = jnp.broadcast_to(att_src2[0, 0][:, None], (_D, _L)).astype(f32)
    Ad2 = jnp.broadcast_to(att_dst2[0, 0][:, None], (_D, _L)).astype(f32)

    R = jnp.concatenate(
        [jnp.kron(eye8, jnp.ones((1, _CH), f32)),
         jnp.zeros((_H, _D), f32)], axis=0)

    zd = jnp.zeros((_K, _D), f32)
    zl = jnp.zeros((_K, _L), f32)
    b1r = b1.reshape(1, _D)
    b2r = b2.reshape(1, _D)

    h1, s1, d1 = _tc_dense(x_pad, W1, As1, Ad1)
    acc1, den1 = _SC_EDGE(src, dst, zd, zl, s1, d1, h1)
    h2, s2, d2 = _tc_combine_dense(acc1, den1, R, b1r, W2, As2, Ad2)
    acc2, den2 = _SC_EDGE(src, dst, zd, zl, s2, d2, h2)
    return _tc_combine_final(acc2, den2, R, b2r)


# speedup vs baseline: 1.1909x; 1.0024x over previous
import functools

import jax
import jax.numpy as jnp
from jax import lax
from jax.experimental import pallas as pl
from jax.experimental.pallas import tpu as pltpu
from jax.experimental.pallas import tpu_sc as plsc

_N = 10000
_D = 128
_H = 8
_CH = 16
_E = 320000

_NC, _NS, _L = 2, 16, 16
_NW = _NC * _NS
_NP = 10240
_RPT = _NP // _NS
_K = 64
_NCHUNK = 162
_EPAD = _NW * _NCHUNK * _K
_EALLOC = _EPAD + _K
_NB = 3


def _sc_edge_build(uniform_e):
    mesh = plsc.VectorSubcoreMesh(
        core_axis_name="c", subcore_axis_name="s",
        num_cores=_NC, num_subcores=_NS)

    @functools.partial(
        pl.kernel,
        out_type=(jax.ShapeDtypeStruct((_NC, _NP, _D), jnp.float32),
                  jax.ShapeDtypeStruct((_NC, _NP, _L), jnp.float32)),
        mesh=mesh,
        compiler_params=pltpu.CompilerParams(use_tc_tiling_on_sc=False),
        scratch_types=[
            pltpu.VMEM((_NB, _K), jnp.int32),
            pltpu.VMEM((_NB, _K), jnp.int32),
            pltpu.VMEM((_NB, _K, _L), jnp.float32),
            pltpu.VMEM((_NB, _K, _L), jnp.float32),
            pltpu.VMEM((_NB, _K, _L), jnp.float32),
            pltpu.VMEM((_NB, _K, _D), jnp.float32),
            pltpu.VMEM_SHARED((_NP, _D), jnp.float32),
            pltpu.VMEM_SHARED((_NP, _L), jnp.float32),
            pltpu.SemaphoreType.DMA((_NB,)),
            pltpu.SemaphoreType.DMA((_NB,)),
        ],
    )
    def sc_edge(src_hbm, dst_hbm, zd_hbm, zl_hbm, as_hbm, ad_hbm, h_hbm,
                acc_out, den_out,
                srcv, dstv, asv, adv, ev, hv, acc_sh, den_sh,
                gsem, ssem):
        cid = lax.axis_index("c")
        sid = lax.axis_index("s")
        wid = cid * _NS + sid
        row0 = sid * _RPT
        tbase = wid * (_NCHUNK * _K)

        pltpu.sync_copy(zd_hbm, hv.at[0])
        pltpu.sync_copy(zl_hbm, ev.at[0])
        for t in range(_RPT // _K):
            pltpu.sync_copy(hv.at[0], acc_sh.at[pl.ds(row0 + t * _K, _K)])
            pltpu.sync_copy(ev.at[0], den_sh.at[pl.ds(row0 + t * _K, _K)])
        plsc.subcore_barrier()

        def issue_gathers(g, b):
            base = tbase + g * _K
            pltpu.sync_copy(src_hbm.at[pl.ds(base, _K)], srcv.at[b])
            pltpu.sync_copy(dst_hbm.at[pl.ds(base, _K)], dstv.at[b])
            pltpu.async_copy(as_hbm.at[srcv.at[b]], asv.at[b], gsem.at[b])
            pltpu.async_copy(ad_hbm.at[dstv.at[b]], adv.at[b], gsem.at[b])
            pltpu.async_copy(h_hbm.at[srcv.at[b]], hv.at[b], gsem.at[b])

        def wait_gathers(b):
            pltpu.make_async_copy(
                as_hbm.at[srcv.at[b]], asv.at[b], gsem.at[b]).wait()
            pltpu.make_async_copy(
                ad_hbm.at[dstv.at[b]], adv.at[b], gsem.at[b]).wait()
            pltpu.make_async_copy(
                h_hbm.at[srcv.at[b]], hv.at[b], gsem.at[b]).wait()

        def wait_scatters(b):
            pltpu.make_async_copy(
                ev.at[b], den_sh.at[dstv.at[b]], ssem.at[b]).wait()
            pltpu.make_async_copy(
                hv.at[b], acc_sh.at[dstv.at[b]], ssem.at[b]).wait()

        issue_gathers(0, 0)

        def superstep(p, carry):
            for b in range(_NB):
                g = _NB * p + b
                nxt = (b + 1) % _NB
                @pl.when(g >= _NB - 1)
                def _():
                    wait_scatters(nxt)
                issue_gathers(g + 1, nxt)
                wait_gathers(b)

                def edge_e(i, c):
                    a = asv[b, i] + adv[b, i]
                    a = jnp.where(a > 0, a, 0.2 * a)
                    ev[b, i] = jnp.exp(a)
                    return c
                lax.fori_loop(0, _K, edge_e, 0)

                def edge_scale(i, c):
                    erow = ev[b, i]
                    for hh in range(_H):
                        if uniform_e:
                            sv = erow
                        else:
                            sv = jnp.full((_L,), erow[hh], jnp.float32)
                        hv[b, i, pl.ds(hh * _L, _L)] = (
                            hv[b, i, pl.ds(hh * _L, _L)] * sv)
                    return c
                lax.fori_loop(0, _K, edge_scale, 0)

                pltpu.async_copy(
                    ev.at[b], den_sh.at[dstv.at[b]], ssem.at[b], add=True)
                pltpu.async_copy(
                    hv.at[b], acc_sh.at[dstv.at[b]], ssem.at[b], add=True)
            return carry
        lax.fori_loop(0, _NCHUNK // _NB, superstep, 0)

        wait_gathers(0)
        wait_scatters(1)
        wait_scatters(2)
        plsc.subcore_barrier()

        for t in range(_RPT // _K):
            pltpu.sync_copy(acc_sh.at[pl.ds(row0 + t * _K, _K)], hv.at[0])
            pltpu.sync_copy(hv.at[0],
                            acc_out.at[cid, pl.ds(row0 + t * _K, _K)])
            pltpu.sync_copy(den_sh.at[pl.ds(row0 + t * _K, _K)], ev.at[0])
            pltpu.sync_copy(ev.at[0],
                            den_out.at[cid, pl.ds(row0 + t * _K, _K)])

    return sc_edge


_SC_EDGE = _sc_edge_build(False)
_SC_EDGE_U = _sc_edge_build(True)


def _tc_dense(xin, W, As, Ad):
    def body(x_ref, w_ref, as_ref, ad_ref, h_ref, s_ref, d_ref):
        h = jnp.dot(x_ref[...], w_ref[...],
                    preferred_element_type=jnp.float32)
        h_ref[...] = h
        s_ref[...] = jnp.dot(h, as_ref[...],
                             preferred_element_type=jnp.float32)
        d_ref[...] = jnp.dot(h, ad_ref[...],
                             preferred_element_type=jnp.float32)
    return pl.pallas_call(
        body,
        out_shape=(jax.ShapeDtypeStruct((_NP, _D), jnp.float32),
                   jax.ShapeDtypeStruct((_NP, _L), jnp.float32),
                   jax.ShapeDtypeStruct((_NP, _L), jnp.float32)),
    )(xin, W, As, Ad)


def _tc_combine_dense(acc, den, R, bias, W, As, Ad):
    def body(a_ref, d_ref, r_ref, b_ref, w_ref, as_ref, ad_ref,
             h_ref, s_ref, dt_ref):
        s = a_ref[0] + a_ref[1]
        dn = d_ref[0] + d_ref[1]
        dd = jnp.dot(dn, r_ref[...], preferred_element_type=jnp.float32)
        o = s / (dd + 1e-16) + b_ref[...]
        o = jnp.where(o > 0, o, jnp.exp(o) - 1.0)
        h = jnp.dot(o, w_ref[...], preferred_element_type=jnp.float32)
        h_ref[...] = h
        s_ref[...] = jnp.dot(h, as_ref[...],
                             preferred_element_type=jnp.float32)
        dt_ref[...] = jnp.dot(h, ad_ref[...],
                              preferred_element_type=jnp.float32)
    return pl.pallas_call(
        body,
        out_shape=(jax.ShapeDtypeStruct((_NP, _D), jnp.float32),
                   jax.ShapeDtypeStruct((_NP, _L), jnp.float32),
                   jax.ShapeDtypeStruct((_NP, _L), jnp.float32)),
    )(acc, den, R, bias, W, As, Ad)


def _tc_combine_final(acc, den, R, bias):
    def body(a_ref, d_ref, r_ref, b_ref, o_ref):
        s = a_ref[0, :_N] + a_ref[1, :_N]
        dn = d_ref[0, :_N] + d_ref[1, :_N]
        dd = jnp.dot(dn, r_ref[...], preferred_element_type=jnp.float32)
        o_ref[...] = s / (dd + 1e-16) + b_ref[...]
    return pl.pallas_call(
        body,
        out_shape=jax.ShapeDtypeStruct((_N, _D), jnp.float32),
    )(acc, den, R, bias)


def kernel(x, edge_index, W1, att_src1, att_dst1, b1,
           W2, att_src2, att_dst2, b2):
    f32 = jnp.float32
    loop = jnp.arange(_N, dtype=edge_index.dtype)
    npad = _EALLOC - (_E + _N)
    padv = jnp.full((npad,), _N, edge_index.dtype)
    src = jnp.concatenate([edge_index[0], loop, padv]).astype(jnp.int32)
    dst = jnp.concatenate([edge_index[1], loop, padv]).astype(jnp.int32)

    x_pad = jnp.zeros((_NP, _D), f32).at[:_N].set(x)

    eye8 = jnp.eye(_H, dtype=f32)
    def blockdiag(att):
        m = (att[0][:, :, None] * eye8[:, None, :]).reshape(_D, _H)
        return jnp.concatenate([m, jnp.zeros((_D, _H), f32)], axis=1)
    As1 = blockdiag(att_src1)
    Ad1 = blockdiag(att_dst1)
    As2 = jnp.broadcast_to(att_src2[0, 0][:, None], (_D, _L)).astype(f32)
    Ad2 = jnp.broadcast_to(att_dst2[0, 0][:, None], (_D, _L)).astype(f32)

    R = jnp.concatenate(
        [jnp.kron(eye8, jnp.ones((1, _CH), f32)),
         jnp.zeros((_H, _D), f32)], axis=0)

    zd = jnp.zeros((_K, _D), f32)
    zl = jnp.zeros((_K, _L), f32)
    b1r = b1.reshape(1, _D)
    b2r = b2.reshape(1, _D)

    h1, s1, d1 = _tc_dense(x_pad, W1, As1, Ad1)
    acc1, den1 = _SC_EDGE(src, dst, zd, zl, s1, d1, h1)
    h2, s2, d2 = _tc_combine_dense(acc1, den1, R, b1r, W2, As2, Ad2)
    acc2, den2 = _SC_EDGE_U(src, dst, zd, zl, s2, d2, h2)
    return _tc_combine_final(acc2, den2, R, b2r)
